# Initial kernel scaffold; baseline (speedup 1.0000x reference)
import jax
import jax.numpy as jnp
from jax.experimental import pallas as pl


def kernel(inputs, targets, cluster_ids):
    raise NotImplementedError("write your pallas kernel here")


# speedup vs baseline: 2.4684x; 2.4684x over previous
import functools

import jax
import jax.numpy as jnp
from jax import lax
from jax.experimental import pallas as pl
from jax.experimental.pallas import tpu as pltpu

_NUM_CLASSES = 5
_EPS = 1e-08
_LANES = 128


def _roll1_flat(a, col_is_last):
    t = jnp.roll(a, -1, axis=1)
    u = jnp.roll(t, -1, axis=0)
    return jnp.where(col_is_last, u, t)


def _loss_kernel(n, R, x_ref, c_ref, y_ref, out_ref,
                 s_ref, sk_ref, yk_ref, ck_ref, acc_ref, wacc_ref):
    planes = [x_ref[i] for i in range(_NUM_CLASSES)]
    m = planes[0]
    for p in planes[1:]:
        m = jnp.maximum(m, p)
    es = [jnp.exp(p - m) for p in planes]
    z = es[0]
    for e in es[1:]:
        z = z + e
    num = jnp.zeros_like(z)
    for i, e in enumerate(es):
        if i:
            num = num + e * jnp.float32(i)
    s = num / z
    s_ref[:] = s

    col = lax.broadcasted_iota(jnp.int32, (R, _LANES), 1)
    row = lax.broadcasted_iota(jnp.int32, (R, _LANES), 0)
    fi = row * _LANES + col
    col_last = col == (_LANES - 1)

    sk_ref[:] = _roll1_flat(s, col_last)
    yk_ref[:] = _roll1_flat(y_ref[:], col_last)
    ck_ref[:] = _roll1_flat(c_ref[:], col_last)
    acc_ref[:] = jnp.zeros((R, _LANES), jnp.float32)
    wacc_ref[:] = jnp.zeros((R, _LANES), jnp.int32)

    def body(carry):
        k, _ = carry
        sk = sk_ref[:]
        yk = yk_ref[:]
        ck = ck_ref[:]
        in_range = fi < (n - k)
        same = (c_ref[:] == ck) & in_range
        yd = y_ref[:] - yk
        active = same & (yd != 0)
        d = (s_ref[:] - sk) * jnp.sign(yd).astype(jnp.float32)
        loss = jnp.log1p(jnp.exp(-d))
        w_dist = jnp.abs(yd).astype(jnp.float32)
        contrib = jnp.where(active, loss * w_dist, 0.0)
        acc_ref[:] = acc_ref[:] + contrib
        wacc_ref[:] = wacc_ref[:] + active.astype(jnp.int32)
        sk_ref[:] = _roll1_flat(sk, col_last)
        yk_ref[:] = _roll1_flat(yk, col_last)
        ck_ref[:] = _roll1_flat(ck, col_last)
        return k + 1, jnp.any(same)

    lax.while_loop(lambda c: c[1], body, (jnp.int32(1), jnp.bool_(True)))

    total = jnp.sum(acc_ref[:])
    w = jnp.sum(wacc_ref[:]).astype(jnp.float32)
    out_ref[0, 0] = jnp.where(jnp.abs(w) < _EPS, 0.0, total / (w + _EPS))


@jax.jit
def kernel(inputs, targets, cluster_ids):
    n = targets.shape[0]
    R = (n + _LANES - 1) // _LANES
    np_ = R * _LANES

    order = jnp.argsort(cluster_ids.astype(jnp.int32))
    cs = jnp.full((np_,), -1, jnp.int32).at[:n].set(
        cluster_ids[order].astype(jnp.int32))
    ys = jnp.zeros((np_,), jnp.int32).at[:n].set(targets[order].astype(jnp.int32))
    xs = jnp.zeros((np_, _NUM_CLASSES), jnp.float32).at[:n].set(
        inputs[order].astype(jnp.float32))

    x_planes = xs.T.reshape(_NUM_CLASSES, R, _LANES)
    cs = cs.reshape(R, _LANES)
    ys = ys.reshape(R, _LANES)

    out = pl.pallas_call(
        functools.partial(_loss_kernel, n, R),
        out_shape=jax.ShapeDtypeStruct((1, 1), jnp.float32),
        in_specs=[
            pl.BlockSpec(memory_space=pltpu.VMEM),
            pl.BlockSpec(memory_space=pltpu.VMEM),
            pl.BlockSpec(memory_space=pltpu.VMEM),
        ],
        out_specs=pl.BlockSpec(memory_space=pltpu.SMEM),
        scratch_shapes=[
            pltpu.VMEM((R, _LANES), jnp.float32),
            pltpu.VMEM((R, _LANES), jnp.float32),
            pltpu.VMEM((R, _LANES), jnp.int32),
            pltpu.VMEM((R, _LANES), jnp.int32),
            pltpu.VMEM((R, _LANES), jnp.float32),
            pltpu.VMEM((R, _LANES), jnp.int32),
        ],
    )(x_planes, cs, ys)
    return out[0, 0]


# speedup vs baseline: 5.2415x; 2.1234x over previous
import functools

import jax
import jax.numpy as jnp
from jax import lax
from jax.experimental import pallas as pl
from jax.experimental.pallas import tpu as pltpu

_NUM_CLASSES = 5
_EPS = 1e-08
_LANES = 128
_SORT_PAD = 131072
_SENTINEL = 1 << 14


def _roll1_flat(a, col_is_last):
    t = jnp.roll(a, -1, axis=1)
    u = jnp.roll(t, -1, axis=0)
    return jnp.where(col_is_last, u, t)


def _pack_kernel(n, R, x_ref, c_ref, y_ref, key_ref, s_ref):
    planes = [x_ref[i] for i in range(_NUM_CLASSES)]
    m = planes[0]
    for p in planes[1:]:
        m = jnp.maximum(m, p)
    es = [jnp.exp(p - m) for p in planes]
    z = es[0]
    for e in es[1:]:
        z = z + e
    num = jnp.zeros_like(z)
    for i, e in enumerate(es):
        if i:
            num = num + e * jnp.float32(i)
    s = num / z

    col = lax.broadcasted_iota(jnp.int32, (R, _LANES), 1)
    row = lax.broadcasted_iota(jnp.int32, (R, _LANES), 0)
    fi = row * _LANES + col
    key = (c_ref[:] << 3) | y_ref[:]
    key_ref[:] = jnp.where(fi < n, key, _SENTINEL)
    s_ref[:] = s


def _loss_kernel(n, R, k_ref, s_ref, out_ref, kk_ref, sk_ref, acc_ref, wacc_ref):
    col = lax.broadcasted_iota(jnp.int32, (R, _LANES), 1)
    row = lax.broadcasted_iota(jnp.int32, (R, _LANES), 0)
    fi = row * _LANES + col
    col_last = col == (_LANES - 1)

    kk_ref[:] = _roll1_flat(k_ref[:], col_last)
    sk_ref[:] = _roll1_flat(s_ref[:], col_last)
    acc_ref[:] = jnp.zeros((R, _LANES), jnp.float32)
    wacc_ref[:] = jnp.zeros((R, _LANES), jnp.int32)

    def body(carry):
        k, _ = carry
        ks = k_ref[:]
        kk = kk_ref[:]
        sk = sk_ref[:]
        in_range = fi < (n - k)
        same = ((ks >> 3) == (kk >> 3)) & in_range
        yd = (ks & 7) - (kk & 7)
        active = same & (yd != 0)
        d = (s_ref[:] - sk) * jnp.sign(yd).astype(jnp.float32)
        loss = jnp.log1p(jnp.exp(-d))
        contrib = jnp.where(active, loss * jnp.abs(yd).astype(jnp.float32), 0.0)
        acc_ref[:] = acc_ref[:] + contrib
        wacc_ref[:] = wacc_ref[:] + active.astype(jnp.int32)
        kk_ref[:] = _roll1_flat(kk, col_last)
        sk_ref[:] = _roll1_flat(sk, col_last)
        return k + 1, jnp.any(same)

    lax.while_loop(lambda c: c[1], body, (jnp.int32(1), jnp.bool_(True)))

    total = jnp.sum(acc_ref[:])
    w = jnp.sum(wacc_ref[:]).astype(jnp.float32)
    out_ref[0, 0] = jnp.where(jnp.abs(w) < _EPS, 0.0, total / (w + _EPS))


@jax.jit
def kernel(inputs, targets, cluster_ids):
    n = targets.shape[0]
    R = (n + _LANES - 1) // _LANES
    np_ = R * _LANES

    xs = jnp.zeros((np_, _NUM_CLASSES), jnp.float32).at[:n].set(
        inputs.astype(jnp.float32))
    x_planes = xs.T.reshape(_NUM_CLASSES, R, _LANES)
    cs = jnp.zeros((np_,), jnp.int32).at[:n].set(
        cluster_ids.astype(jnp.int32)).reshape(R, _LANES)
    ys = jnp.zeros((np_,), jnp.int32).at[:n].set(
        targets.astype(jnp.int32)).reshape(R, _LANES)

    key, sval = pl.pallas_call(
        functools.partial(_pack_kernel, n, R),
        out_shape=(
            jax.ShapeDtypeStruct((R, _LANES), jnp.int32),
            jax.ShapeDtypeStruct((R, _LANES), jnp.float32),
        ),
        in_specs=[
            pl.BlockSpec(memory_space=pltpu.VMEM),
            pl.BlockSpec(memory_space=pltpu.VMEM),
            pl.BlockSpec(memory_space=pltpu.VMEM),
        ],
        out_specs=(
            pl.BlockSpec(memory_space=pltpu.VMEM),
            pl.BlockSpec(memory_space=pltpu.VMEM),
        ),
    )(x_planes, cs, ys)

    pad = _SORT_PAD - np_
    keys_flat = jnp.concatenate(
        [key.reshape(-1), jnp.full((pad,), _SENTINEL, jnp.int32)])
    vals_flat = jnp.concatenate(
        [sval.reshape(-1).view(jnp.int32), jnp.zeros((pad,), jnp.int32)])
    keys_sorted, vals_sorted = lax.sort((keys_flat, vals_flat), num_keys=1)
    ks = keys_sorted[:np_].reshape(R, _LANES)
    ss = vals_sorted[:np_].view(jnp.float32).reshape(R, _LANES)

    out = pl.pallas_call(
        functools.partial(_loss_kernel, n, R),
        out_shape=jax.ShapeDtypeStruct((1, 1), jnp.float32),
        in_specs=[
            pl.BlockSpec(memory_space=pltpu.VMEM),
            pl.BlockSpec(memory_space=pltpu.VMEM),
        ],
        out_specs=pl.BlockSpec(memory_space=pltpu.SMEM),
        scratch_shapes=[
            pltpu.VMEM((R, _LANES), jnp.int32),
            pltpu.VMEM((R, _LANES), jnp.float32),
            pltpu.VMEM((R, _LANES), jnp.float32),
            pltpu.VMEM((R, _LANES), jnp.int32),
        ],
    )(ks, ss)
    return out[0, 0]
